# baseline (speedup 1.0000x reference)
import functools

import jax
import jax.numpy as jnp
from jax import lax
from jax.experimental import pallas as pl
from jax.experimental.pallas import tpu as pltpu
from jax.experimental.pallas import tpu_sc as plsc

_N = 16384
_NRAYS = 1024
_SR = 4
_K = 8
_EMB = 32
_RADIUS = 0.35
_Q = _NRAYS * _SR
_BQ = 128
_NB = _Q // _BQ
_DF = 3 + _EMB + 3 + 3
_DP = 48
_R2 = _RADIUS * _RADIUS


def _knn_body(q_ref, pt_ref, gidx_ref, pidx_ref, mask_ref):
    q = q_ref[...]
    pt = pt_ref[...]
    q2 = jnp.sum(q * q, axis=1, keepdims=True)
    p2 = jnp.sum(pt * pt, axis=0, keepdims=True)
    dot = lax.dot_general(q, pt, (((1,), (0,)), ((), ())),
                          preferred_element_type=jnp.float32)
    vals = q2 + p2 - 2.0 * dot
    iotaf = lax.broadcasted_iota(jnp.int32, (_BQ, _N), 1).astype(jnp.float32)
    row = (pl.program_id(0) * _BQ
           + lax.broadcasted_iota(jnp.int32, (_BQ, 1), 0))
    for j in range(_K):
        m = jnp.min(vals, axis=1, keepdims=True)
        idxf = jnp.min(jnp.where(vals == m, iotaf, jnp.float32(_N)),
                       axis=1, keepdims=True)
        idx = idxf.astype(jnp.int32)
        ok = m <= _R2
        mask_ref[:, j:j + 1] = ok.astype(jnp.int32)
        pidx_ref[:, j:j + 1] = jnp.where(ok, idx, -1)
        pad = _N + ((row * _K + j) & 63)
        gidx_ref[:, j:j + 1] = jnp.where(ok, idx, pad)
        if j < _K - 1:
            vals = jnp.where(iotaf == idxf, jnp.float32(jnp.inf), vals)


def _knn(q, pt, nrows):
    return pl.pallas_call(
        _knn_body,
        grid=(nrows // _BQ,),
        in_specs=[pl.BlockSpec((_BQ, 3), lambda i: (i, 0)),
                  pl.BlockSpec((3, _N), lambda i: (0, 0))],
        out_specs=[pl.BlockSpec((_BQ, _K), lambda i: (i, 0)),
                   pl.BlockSpec((_BQ, _K), lambda i: (i, 0)),
                   pl.BlockSpec((_BQ, _K), lambda i: (i, 0))],
        out_shape=[jax.ShapeDtypeStruct((nrows, _K), jnp.int32),
                   jax.ShapeDtypeStruct((nrows, _K), jnp.int32),
                   jax.ShapeDtypeStruct((nrows, _K), jnp.int32)],
    )(q, pt)


@functools.lru_cache(maxsize=4)
def _make_gather(b):
    info = plsc.get_sparse_core_info()
    nc, ns = info.num_cores, info.num_subcores
    nw = nc * ns
    b_per_w = b // nw
    ch = 128
    nch = b_per_w // ch
    mesh = plsc.VectorSubcoreMesh(core_axis_name="c", subcore_axis_name="s")

    @functools.partial(
        pl.kernel, mesh=mesh,
        out_type=jax.ShapeDtypeStruct((b, _DP), jnp.float32),
        scratch_types=[pltpu.VMEM((nch, ch), jnp.int32),
                       pltpu.VMEM((b_per_w, _DP), jnp.float32),
                       pltpu.SemaphoreType.DMA],
        compiler_params=pltpu.CompilerParams(use_tc_tiling_on_sc=False),
    )
    def gk(table_hbm, idx_hbm, out_hbm, idx_v, rows_v, sem):
        wid = lax.axis_index("s") * nc + lax.axis_index("c")
        pltpu.sync_copy(idx_hbm.at[wid], idx_v)
        copies = [pltpu.async_copy(table_hbm.at[idx_v.at[j]],
                                   rows_v.at[pl.ds(j * ch, ch)], sem)
                  for j in range(nch)]
        for c in copies:
            c.wait()
        pltpu.sync_copy(rows_v, out_hbm.at[pl.ds(wid * b_per_w, b_per_w)])

    return gk, nw, nch, ch


def kernel(points_pos, points_emb, points_color, points_dir,
           raydir, camrotc2w, campos, near, far):
    d = raydir / (jnp.linalg.norm(raydir, axis=-1, keepdims=True) + 1e-8)
    dirs_w = d @ camrotc2w.T
    t = jnp.linspace(near[0], far[0], _SR)
    sample_loc_w = campos[None, None, :] + t[None, :, None] * dirs_w[:, None, :]
    q = sample_loc_w.reshape(-1, 3)

    table = jnp.concatenate(
        [points_pos, points_emb, points_color, points_dir,
         jnp.zeros((_N, _DP - _DF), jnp.float32)], axis=1)
    table = jnp.concatenate([table, jnp.zeros((64, _DP), jnp.float32)], axis=0)

    pt = points_pos.T
    half = _Q // 2
    gather, nw, nch, ch = _make_gather(half * _K)
    rows, pidxs, masks = [], [], []
    for h in range(2):
        qh = lax.slice_in_dim(q, h * half, (h + 1) * half, axis=0)
        gidx, pidx, maski = _knn(qh, pt, half)
        rows.append(gather(table, gidx.reshape(nw, nch, ch)))
        pidxs.append(pidx)
        masks.append(maski)

    rows = jnp.concatenate(rows, axis=0)
    pidx = jnp.concatenate(pidxs, axis=0)
    maski = jnp.concatenate(masks, axis=0)
    sampled_feat = rows[:, :_DF].reshape(_NRAYS, _SR, _K, _DF)
    sample_pidx = pidx.reshape(_NRAYS, _SR, _K)
    sample_pnt_mask = (maski != 0).reshape(_NRAYS, _SR, _K)
    return sampled_feat, sample_pidx, sample_loc_w, sample_pnt_mask


# speedup vs baseline: 1.0330x; 1.0330x over previous
import functools

import jax
import jax.numpy as jnp
from jax import lax
from jax.experimental import pallas as pl
from jax.experimental.pallas import tpu as pltpu
from jax.experimental.pallas import tpu_sc as plsc

_N = 16384
_NRAYS = 1024
_SR = 4
_K = 8
_EMB = 32
_RADIUS = 0.35
_Q = _NRAYS * _SR
_BQ = 128
_NB = _Q // _BQ
_DF = 3 + _EMB + 3 + 3
_DP = 48
_R2 = _RADIUS * _RADIUS


def _knn_body(q_ref, pt_ref, gidx_ref, pidx_ref, mask_ref):
    q = q_ref[...]
    pt = pt_ref[...]
    q2 = jnp.sum(q * q, axis=1, keepdims=True)
    p2 = jnp.sum(pt * pt, axis=0, keepdims=True)
    dot = lax.dot_general(q, pt, (((1,), (0,)), ((), ())),
                          preferred_element_type=jnp.float32)
    vals = q2 + p2 - 2.0 * dot
    iotaf = lax.broadcasted_iota(jnp.int32, (_BQ, _N), 1).astype(jnp.float32)
    row = (pl.program_id(0) * _BQ
           + lax.broadcasted_iota(jnp.int32, (_BQ, 1), 0))
    for j in range(_K):
        m = jnp.min(vals, axis=1, keepdims=True)
        idxf = jnp.min(jnp.where(vals == m, iotaf, jnp.float32(_N)),
                       axis=1, keepdims=True)
        idx = idxf.astype(jnp.int32)
        ok = m <= _R2
        mask_ref[:, j:j + 1] = ok.astype(jnp.int32)
        pidx_ref[:, j:j + 1] = jnp.where(ok, idx, -1)
        pad = _N + ((row * _K + j) & 63)
        gidx_ref[:, j:j + 1] = jnp.where(ok, idx, pad)
        if j < _K - 1:
            vals = jnp.where(iotaf == idxf, jnp.float32(jnp.inf), vals)


def _knn(q, pt, nrows):
    return pl.pallas_call(
        _knn_body,
        grid=(nrows // _BQ,),
        in_specs=[pl.BlockSpec((_BQ, 3), lambda i: (i, 0)),
                  pl.BlockSpec((3, _N), lambda i: (0, 0))],
        out_specs=[pl.BlockSpec((_BQ, _K), lambda i: (i, 0)),
                   pl.BlockSpec((_BQ, _K), lambda i: (i, 0)),
                   pl.BlockSpec((_BQ, _K), lambda i: (i, 0))],
        out_shape=[jax.ShapeDtypeStruct((nrows, _K), jnp.int32),
                   jax.ShapeDtypeStruct((nrows, _K), jnp.int32),
                   jax.ShapeDtypeStruct((nrows, _K), jnp.int32)],
    )(q, pt)


@functools.lru_cache(maxsize=4)
def _make_gather(b):
    info = plsc.get_sparse_core_info()
    nc, ns = info.num_cores, info.num_subcores
    nw = nc * ns
    b_per_w = b // nw
    ch = 128
    nch = b_per_w // ch
    mesh = plsc.VectorSubcoreMesh(core_axis_name="c", subcore_axis_name="s")

    @functools.partial(
        pl.kernel, mesh=mesh,
        out_type=jax.ShapeDtypeStruct((b, _DP), jnp.float32),
        scratch_types=[pltpu.VMEM((nch, ch), jnp.int32),
                       pltpu.VMEM((b_per_w, _DP), jnp.float32),
                       pltpu.SemaphoreType.DMA],
        compiler_params=pltpu.CompilerParams(use_tc_tiling_on_sc=False),
    )
    def gk(table_hbm, idx_hbm, out_hbm, idx_v, rows_v, sem):
        wid = lax.axis_index("s") * nc + lax.axis_index("c")
        pltpu.sync_copy(idx_hbm.at[wid], idx_v)
        copies = [pltpu.async_copy(table_hbm.at[idx_v.at[j]],
                                   rows_v.at[pl.ds(j * ch, ch)], sem)
                  for j in range(nch)]
        for c in copies:
            c.wait()
        pltpu.sync_copy(rows_v, out_hbm.at[pl.ds(wid * b_per_w, b_per_w)])

    return gk, nw, nch, ch


def kernel(points_pos, points_emb, points_color, points_dir,
           raydir, camrotc2w, campos, near, far):
    d = raydir / (jnp.linalg.norm(raydir, axis=-1, keepdims=True) + 1e-8)
    dirs_w = d @ camrotc2w.T
    t = jnp.linspace(near[0], far[0], _SR)
    sample_loc_w = campos[None, None, :] + t[None, :, None] * dirs_w[:, None, :]
    q = sample_loc_w.reshape(-1, 3)

    table = jnp.concatenate(
        [points_pos, points_emb, points_color, points_dir,
         jnp.zeros((_N, _DP - _DF), jnp.float32)], axis=1)
    table = jnp.concatenate([table, jnp.zeros((64, _DP), jnp.float32)], axis=0)

    gidx, pidx, maski = _knn(q, points_pos.T, _Q)
    gather, nw, nch, ch = _make_gather(_Q * _K)
    rows = gather(table, gidx.reshape(nw, nch, ch))
    sampled_feat = rows[:, :_DF].reshape(_NRAYS, _SR, _K, _DF)
    sample_pidx = pidx.reshape(_NRAYS, _SR, _K)
    sample_pnt_mask = (maski != 0).reshape(_NRAYS, _SR, _K)
    return sampled_feat, sample_pidx, sample_loc_w, sample_pnt_mask


# speedup vs baseline: 1.0694x; 1.0353x over previous
import functools

import jax
import jax.numpy as jnp
from jax import lax
from jax.experimental import pallas as pl
from jax.experimental.pallas import tpu as pltpu
from jax.experimental.pallas import tpu_sc as plsc

_N = 16384
_NRAYS = 1024
_SR = 4
_K = 8
_EMB = 32
_RADIUS = 0.35
_Q = _NRAYS * _SR
_BQ = 256
_NB = _Q // _BQ
_DF = 3 + _EMB + 3 + 3
_DP = 48
_R2 = _RADIUS * _RADIUS


def _knn_body(q_ref, pt_ref, gidx_ref, pidx_ref, mask_ref):
    q = q_ref[...]
    pt = pt_ref[...]
    q2 = jnp.sum(q * q, axis=1, keepdims=True)
    p2 = jnp.sum(pt * pt, axis=0, keepdims=True)
    dot = lax.dot_general(q, pt, (((1,), (0,)), ((), ())),
                          preferred_element_type=jnp.float32)
    vals = q2 + p2 - 2.0 * dot
    iotaf = lax.broadcasted_iota(jnp.int32, (_BQ, _N), 1).astype(jnp.float32)
    row = (pl.program_id(0) * _BQ
           + lax.broadcasted_iota(jnp.int32, (_BQ, 1), 0))
    for j in range(_K):
        m = jnp.min(vals, axis=1, keepdims=True)
        idxf = jnp.min(jnp.where(vals == m, iotaf, jnp.float32(_N)),
                       axis=1, keepdims=True)
        idx = idxf.astype(jnp.int32)
        ok = m <= _R2
        mask_ref[:, j:j + 1] = ok.astype(jnp.int32)
        pidx_ref[:, j:j + 1] = jnp.where(ok, idx, -1)
        pad = _N + ((row * _K + j) & 63)
        gidx_ref[:, j:j + 1] = jnp.where(ok, idx, pad)
        if j < _K - 1:
            vals = jnp.where(iotaf == idxf, jnp.float32(jnp.inf), vals)


def _knn(q, pt, nrows):
    return pl.pallas_call(
        _knn_body,
        grid=(nrows // _BQ,),
        in_specs=[pl.BlockSpec((_BQ, 3), lambda i: (i, 0)),
                  pl.BlockSpec((3, _N), lambda i: (0, 0))],
        out_specs=[pl.BlockSpec((_BQ, _K), lambda i: (i, 0)),
                   pl.BlockSpec((_BQ, _K), lambda i: (i, 0)),
                   pl.BlockSpec((_BQ, _K), lambda i: (i, 0))],
        out_shape=[jax.ShapeDtypeStruct((nrows, _K), jnp.int32),
                   jax.ShapeDtypeStruct((nrows, _K), jnp.int32),
                   jax.ShapeDtypeStruct((nrows, _K), jnp.int32)],
    )(q, pt)


@functools.lru_cache(maxsize=4)
def _make_gather(b):
    info = plsc.get_sparse_core_info()
    nc, ns = info.num_cores, info.num_subcores
    nw = nc * ns
    b_per_w = b // nw
    ch = 128
    nch = b_per_w // ch
    mesh = plsc.VectorSubcoreMesh(core_axis_name="c", subcore_axis_name="s")

    @functools.partial(
        pl.kernel, mesh=mesh,
        out_type=jax.ShapeDtypeStruct((b, _DP), jnp.float32),
        scratch_types=[pltpu.VMEM((nch, ch), jnp.int32),
                       pltpu.VMEM((b_per_w, _DP), jnp.float32),
                       pltpu.SemaphoreType.DMA],
        compiler_params=pltpu.CompilerParams(use_tc_tiling_on_sc=False),
    )
    def gk(table_hbm, idx_hbm, out_hbm, idx_v, rows_v, sem):
        wid = lax.axis_index("s") * nc + lax.axis_index("c")
        pltpu.sync_copy(idx_hbm.at[wid], idx_v)
        copies = [pltpu.async_copy(table_hbm.at[idx_v.at[j]],
                                   rows_v.at[pl.ds(j * ch, ch)], sem)
                  for j in range(nch)]
        for c in copies:
            c.wait()
        pltpu.sync_copy(rows_v, out_hbm.at[pl.ds(wid * b_per_w, b_per_w)])

    return gk, nw, nch, ch


def kernel(points_pos, points_emb, points_color, points_dir,
           raydir, camrotc2w, campos, near, far):
    d = raydir / (jnp.linalg.norm(raydir, axis=-1, keepdims=True) + 1e-8)
    dirs_w = d @ camrotc2w.T
    t = jnp.linspace(near[0], far[0], _SR)
    sample_loc_w = campos[None, None, :] + t[None, :, None] * dirs_w[:, None, :]
    q = sample_loc_w.reshape(-1, 3)

    table = jnp.concatenate(
        [points_pos, points_emb, points_color, points_dir,
         jnp.zeros((_N, _DP - _DF), jnp.float32)], axis=1)
    table = jnp.concatenate([table, jnp.zeros((64, _DP), jnp.float32)], axis=0)

    gidx, pidx, maski = _knn(q, points_pos.T, _Q)
    gather, nw, nch, ch = _make_gather(_Q * _K)
    rows = gather(table, gidx.reshape(nw, nch, ch))
    sampled_feat = rows[:, :_DF].reshape(_NRAYS, _SR, _K, _DF)
    sample_pidx = pidx.reshape(_NRAYS, _SR, _K)
    sample_pnt_mask = (maski != 0).reshape(_NRAYS, _SR, _K)
    return sampled_feat, sample_pidx, sample_loc_w, sample_pnt_mask
